# baseline (speedup 1.0000x reference)
import dataclasses
import functools

import jax
import jax.numpy as jnp
from jax import lax
from jax.experimental import pallas as pl
from jax.experimental.pallas import tpu as pltpu
from jax.experimental.pallas import tpu_sc as plsc

VOCAB = 1000000
DIM = 64
BATCH = 16384
NEG = 5
NSCORE = NEG + 1

NC = 2
NS = 16
LANES = 16
NW = NC * NS
BPW = BATCH // NW
CHUNK = 64
T = BPW // CHUNK
NBLK = VOCAB // 2


def _sc_scores(pos_u, pos_v, neg_t, u3, v3):
    mesh = plsc.VectorSubcoreMesh(core_axis_name="c", subcore_axis_name="s")
    cp = pltpu.CompilerParams()
    if "needs_layout_passes" in getattr(pltpu.CompilerParams, "__dataclass_fields__", {}):
        cp = dataclasses.replace(cp, needs_layout_passes=False)

    @functools.partial(
        pl.kernel,
        mesh=mesh,
        out_type=jax.ShapeDtypeStruct((NW * T * NSCORE * CHUNK,), jnp.float32),
        scratch_types=[
            pltpu.VMEM((CHUNK,), jnp.int32),
            pltpu.VMEM((CHUNK,), jnp.int32),
            pltpu.VMEM((NEG * CHUNK,), jnp.int32),
            pltpu.VMEM((CHUNK,), jnp.int32),
            pltpu.VMEM((CHUNK,), jnp.int32),
            pltpu.VMEM((NEG * CHUNK,), jnp.int32),
            pltpu.VMEM((CHUNK, 2 * DIM), jnp.float32),
            pltpu.VMEM((CHUNK, 2 * DIM), jnp.float32),
            pltpu.VMEM((NEG, CHUNK, 2 * DIM), jnp.float32),
            pltpu.VMEM((NSCORE * CHUNK,), jnp.float32),
            pltpu.SemaphoreType.DMA,
        ],
        compiler_params=cp,
    )
    def scores_kernel(pos_u_hbm, pos_v_hbm, neg_hbm, u_w_hbm, v_w_hbm, out_hbm,
                      u_idx, v_idx, n_idx, u_blk, v_blk, n_blk,
                      u_rows, v_rows, n_rows, s_buf, sem):
        wid = lax.axis_index("s") * NC + lax.axis_index("c")

        @pl.loop(0, T)
        def _chunk(t):
            off = wid * BPW + t * CHUNK
            pltpu.sync_copy(pos_u_hbm.at[pl.ds(off, CHUNK)], u_idx)
            pltpu.sync_copy(pos_v_hbm.at[pl.ds(off, CHUNK)], v_idx)
            for k in range(NEG):
                pltpu.sync_copy(neg_hbm.at[pl.ds(k * BATCH + off, CHUNK)],
                                n_idx.at[pl.ds(k * CHUNK, CHUNK)])

            for c0 in range(0, CHUNK, LANES):
                sl = pl.ds(c0, LANES)
                u_blk[sl] = u_idx[sl] >> 1
                v_blk[sl] = v_idx[sl] >> 1
            for c0 in range(0, NEG * CHUNK, LANES):
                sl = pl.ds(c0, LANES)
                n_blk[sl] = n_idx[sl] >> 1

            copies = [
                pltpu.async_copy(u_w_hbm.at[u_blk], u_rows, sem),
                pltpu.async_copy(v_w_hbm.at[v_blk], v_rows, sem),
            ]
            for k in range(NEG):
                copies.append(
                    pltpu.async_copy(v_w_hbm.at[n_blk.at[pl.ds(k * CHUNK, CHUNK)]],
                                     n_rows.at[k], sem))
            for c in copies:
                c.wait()

            base_iota = lax.iota(jnp.int32, LANES)
            for g in range(CHUNK // LANES):
                row = base_iota + (g * LANES)
                sub_u = (u_idx[pl.ds(g * LANES, LANES)] & 1) * DIM
                sub_v = (v_idx[pl.ds(g * LANES, LANES)] & 1) * DIM
                sub_n = [(n_idx[pl.ds(k * CHUNK + g * LANES, LANES)] & 1) * DIM
                         for k in range(NEG)]

                def dbody(dd, accs, row=row, sub_u=sub_u, sub_v=sub_v, sub_n=sub_n):
                    col = jnp.full((LANES,), dd, jnp.int32)
                    u_col = plsc.load_gather(u_rows, [row, sub_u + col])
                    v_col = plsc.load_gather(v_rows, [row, sub_v + col])
                    new = [accs[0] + u_col * v_col]
                    for k in range(NEG):
                        kk = jnp.full((LANES,), k, jnp.int32)
                        n_col = plsc.load_gather(n_rows, [kk, row, sub_n[k] + col])
                        new.append(accs[1 + k] + u_col * n_col)
                    return tuple(new)

                accs = tuple(jnp.zeros((LANES,), jnp.float32) for _ in range(NSCORE))
                accs = lax.fori_loop(0, DIM, dbody, accs)
                s_buf[pl.ds(g * LANES, LANES)] = accs[0]
                for k in range(NEG):
                    s_buf[pl.ds((1 + k) * CHUNK + g * LANES, LANES)] = -accs[1 + k]

            pltpu.sync_copy(
                s_buf,
                out_hbm.at[pl.ds((wid * T + t) * NSCORE * CHUNK, NSCORE * CHUNK)])

    return scores_kernel(pos_u, pos_v, neg_t, u3, v3)


def _tc_loss(scores2d):
    def body(x_ref, o_ref):
        s = x_ref[...]
        y = jnp.minimum(s, 0.0) - jnp.log1p(jnp.exp(-jnp.abs(s)))
        o_ref[0, 0] = -jnp.sum(y)

    return pl.pallas_call(
        body,
        out_shape=jax.ShapeDtypeStruct((1, 1), jnp.float32),
        out_specs=pl.BlockSpec(memory_space=pltpu.SMEM),
    )(scores2d)


def kernel(pos_u, pos_v, neg_v, u_weight, v_weight):
    pos_u = pos_u.astype(jnp.int32)
    pos_v = pos_v.astype(jnp.int32)
    neg_t = neg_v.astype(jnp.int32).T.reshape(-1)
    u3 = jnp.concatenate([u_weight[0::2], u_weight[1::2]], axis=1)
    v3 = jnp.concatenate([v_weight[0::2], v_weight[1::2]], axis=1)
    scores = _sc_scores(pos_u, pos_v, neg_t, u3, v3)
    loss = _tc_loss(scores.reshape(NW * T * NSCORE * CHUNK // 128, 128))
    return loss[0, 0]


# speedup vs baseline: 20.9670x; 20.9670x over previous
import dataclasses
import functools

import jax
import jax.numpy as jnp
from jax import lax
from jax.experimental import pallas as pl
from jax.experimental.pallas import tpu as pltpu
from jax.experimental.pallas import tpu_sc as plsc

VOCAB = 1000000
DIM = 64
BATCH = 16384
NEG = 5
NSCORE = NEG + 1
NSTREAM = NEG + 2

NC = 2
NS = 16
LANES = 16
NW = NC * NS
BPW = BATCH // NW
CHUNK = 128
T = BPW // CHUNK


def _sc_scores(pos_u, pos_v, neg_t, u_w, v_w):
    mesh = plsc.VectorSubcoreMesh(core_axis_name="c", subcore_axis_name="s")
    cp = pltpu.CompilerParams()
    if "needs_layout_passes" in getattr(pltpu.CompilerParams, "__dataclass_fields__", {}):
        cp = dataclasses.replace(cp, needs_layout_passes=False)

    @functools.partial(
        pl.kernel,
        mesh=mesh,
        out_type=jax.ShapeDtypeStruct((NW * T * NSCORE * CHUNK,), jnp.float32),
        scratch_types=[
            pltpu.SMEM((NSTREAM * CHUNK,), jnp.int32),
            pltpu.VMEM((NSTREAM * CHUNK,), jnp.int32),
            pltpu.VMEM_SHARED((NS, NSTREAM * CHUNK), jnp.int32),
            pltpu.VMEM((CHUNK, DIM), jnp.float32),
            pltpu.VMEM((CHUNK, DIM), jnp.float32),
            pltpu.VMEM((NEG, CHUNK, DIM), jnp.float32),
            pltpu.VMEM((NSCORE * CHUNK,), jnp.float32),
            pltpu.SemaphoreType.DMA,
            pltpu.SemaphoreType.DMA,
        ],
        compiler_params=cp,
    )
    def scores_kernel(pos_u_hbm, pos_v_hbm, neg_hbm, u_w_hbm, v_w_hbm, out_hbm,
                      idx_s, idx_v, idx_sp, u_rows, v_rows, n_rows, s_buf,
                      sem, sem2):
        sid = lax.axis_index("s")
        wid = sid * NC + lax.axis_index("c")

        @pl.loop(0, T)
        def _chunk(t):
            off = wid * BPW + t * CHUNK
            idx_copies = [
                pltpu.async_copy(pos_u_hbm.at[pl.ds(off, CHUNK)],
                                 idx_v.at[pl.ds(0, CHUNK)], sem2),
                pltpu.async_copy(pos_v_hbm.at[pl.ds(off, CHUNK)],
                                 idx_v.at[pl.ds(CHUNK, CHUNK)], sem2),
            ]
            for k in range(NEG):
                idx_copies.append(
                    pltpu.async_copy(neg_hbm.at[pl.ds(k * BATCH + off, CHUNK)],
                                     idx_v.at[pl.ds((2 + k) * CHUNK, CHUNK)], sem2))
            for c in idx_copies:
                c.wait()
            pltpu.sync_copy(idx_v, idx_sp.at[sid])
            pltpu.sync_copy(idx_sp.at[sid], idx_s)

            @pl.loop(0, CHUNK)
            def _row(i):
                r_u = idx_s[i]
                pltpu.async_copy(u_w_hbm.at[pl.ds(r_u, 1), :],
                                 u_rows.at[pl.ds(i, 1), :], sem)
                r_v = idx_s[CHUNK + i]
                pltpu.async_copy(v_w_hbm.at[pl.ds(r_v, 1), :],
                                 v_rows.at[pl.ds(i, 1), :], sem)
                for k in range(NEG):
                    r_n = idx_s[(2 + k) * CHUNK + i]
                    pltpu.async_copy(v_w_hbm.at[pl.ds(r_n, 1), :],
                                     n_rows.at[k, pl.ds(i, 1), :], sem)

            pltpu.make_async_copy(u_w_hbm.at[pl.ds(0, CHUNK), :], u_rows, sem).wait()
            pltpu.make_async_copy(v_w_hbm.at[pl.ds(0, CHUNK), :], v_rows, sem).wait()
            for k in range(NEG):
                pltpu.make_async_copy(v_w_hbm.at[pl.ds(0, CHUNK), :],
                                      n_rows.at[k], sem).wait()

            base_iota = lax.iota(jnp.int32, LANES)
            for g in range(CHUNK // LANES):
                row = base_iota + (g * LANES)

                def dbody(dd, accs, row=row):
                    col = jnp.full((LANES,), dd, jnp.int32)
                    u_col = plsc.load_gather(u_rows, [row, col])
                    v_col = plsc.load_gather(v_rows, [row, col])
                    new = [accs[0] + u_col * v_col]
                    for k in range(NEG):
                        kk = jnp.full((LANES,), k, jnp.int32)
                        n_col = plsc.load_gather(n_rows, [kk, row, col])
                        new.append(accs[1 + k] + u_col * n_col)
                    return tuple(new)

                accs = tuple(jnp.zeros((LANES,), jnp.float32) for _ in range(NSCORE))
                accs = lax.fori_loop(0, DIM, dbody, accs)
                s_buf[pl.ds(g * LANES, LANES)] = accs[0]
                for k in range(NEG):
                    s_buf[pl.ds((1 + k) * CHUNK + g * LANES, LANES)] = -accs[1 + k]

            pltpu.sync_copy(
                s_buf,
                out_hbm.at[pl.ds((wid * T + t) * NSCORE * CHUNK, NSCORE * CHUNK)])

    return scores_kernel(pos_u, pos_v, neg_t, u_w, v_w)


def _tc_loss(scores2d):
    def body(x_ref, o_ref):
        s = x_ref[...]
        y = jnp.minimum(s, 0.0) - jnp.log1p(jnp.exp(-jnp.abs(s)))
        o_ref[0, 0] = -jnp.sum(y)

    return pl.pallas_call(
        body,
        out_shape=jax.ShapeDtypeStruct((1, 1), jnp.float32),
        out_specs=pl.BlockSpec(memory_space=pltpu.SMEM),
    )(scores2d)


def kernel(pos_u, pos_v, neg_v, u_weight, v_weight):
    pos_u = pos_u.astype(jnp.int32)
    pos_v = pos_v.astype(jnp.int32)
    neg_t = neg_v.astype(jnp.int32).T.reshape(-1)
    scores = _sc_scores(pos_u, pos_v, neg_t, u_weight, v_weight)
    loss = _tc_loss(scores.reshape(NW * T * NSCORE * CHUNK // 128, 128))
    return loss[0, 0]
